# baseline (speedup 1.0000x reference)
import jax
import jax.numpy as jnp
from jax.experimental import pallas as pl
from jax.experimental.pallas import tpu as pltpu

_HF = 256
_HO = _HF * 4


_G = 4


def _rf_body(v_ref, eh_ref, e768_ref, mhl_ref, mhr_ref, o_ref):
    mhl = jnp.concatenate([mhl_ref[...], mhl_ref[...]], axis=1)
    mhr = jnp.concatenate([mhr_ref[...], mhr_ref[...]], axis=1)
    for g in range(_G):
        v = jnp.maximum(v_ref[g], 0.0).astype(jnp.bfloat16)
        u = jnp.dot(eh_ref[...], v, preferred_element_type=jnp.float32)
        uu = jnp.concatenate([u[:4], u[:-4]], axis=0)
        ud = jnp.concatenate([u[4:], u[-4:]], axis=0)
        oh = jnp.maximum(jnp.maximum(u, uu * mhl), ud * mhr)
        left = jnp.concatenate([oh[:, :1], oh[:, :-1]], axis=1)
        right = jnp.concatenate([oh[:, 1:], oh[:, -1:]], axis=1)
        a = jnp.maximum(left, oh)
        b = jnp.maximum(oh, right)
        c = jnp.maximum(a, right)
        lhs = jnp.concatenate([a, c, b], axis=1).astype(jnp.bfloat16)
        o_ref[g] = jnp.dot(lhs, e768_ref[...],
                           preferred_element_type=jnp.float32)


def kernel(inputs):
    bsz = inputs.shape[0]
    nmaps = bsz * 3
    v8 = jax.lax.optimization_barrier(inputs[..., :8])
    v = jnp.transpose(v8[..., :3], (0, 3, 1, 2)).reshape(nmaps, _HF, _HF)
    i = jnp.arange(_HF, dtype=jnp.int32)
    p = jnp.arange(_HO, dtype=jnp.int32)
    q, r = p // 4, p % 4
    ehm = (q[:, None] == i[None, :]).astype(jnp.bfloat16)
    src = jnp.where(r == 0, q, jnp.where(r == 3, 512 + q, 256 + q))
    e768 = (jnp.arange(768, dtype=jnp.int32)[:, None] == src[None, :]
            ).astype(jnp.bfloat16)
    mhl = jnp.tile((r <= 2).astype(jnp.float32)[:, None], (1, 128))
    mhr = jnp.tile((r >= 1).astype(jnp.float32)[:, None], (1, 128))

    out = pl.pallas_call(
        _rf_body,
        grid=(nmaps // _G,),
        in_specs=[
            pl.BlockSpec((_G, _HF, _HF), lambda m: (m, 0, 0)),
            pl.BlockSpec((_HO, _HF), lambda m: (0, 0)),
            pl.BlockSpec((768, _HO), lambda m: (0, 0)),
            pl.BlockSpec((_HO, 128), lambda m: (0, 0)),
            pl.BlockSpec((_HO, 128), lambda m: (0, 0)),
        ],
        out_specs=pl.BlockSpec((_G, _HO, _HO), lambda m: (m, 0, 0)),
        out_shape=jax.ShapeDtypeStruct((nmaps, _HO, _HO), jnp.float32),
        compiler_params=pltpu.CompilerParams(
            dimension_semantics=("parallel",),
        ),
    )(v, ehm, e768, mhl, mhr)
    return out.reshape(bsz, 3, _HO, _HO)


# speedup vs baseline: 1.0070x; 1.0070x over previous
import jax
import jax.numpy as jnp
from jax.experimental import pallas as pl
from jax.experimental.pallas import tpu as pltpu

_HF = 256
_HO = _HF * 4


_G = 4


def _rf_body(v_ref, eh_ref, e768_ref, mhl_ref, mhr_ref, o_ref):
    mhl = jnp.concatenate([mhl_ref[...], mhl_ref[...]], axis=1)
    mhr = jnp.concatenate([mhr_ref[...], mhr_ref[...]], axis=1)
    for g in range(_G):
        v = jnp.maximum(v_ref[g], 0.0).astype(jnp.bfloat16)
        u = jnp.dot(eh_ref[...], v, preferred_element_type=jnp.float32)
        uu = jnp.concatenate([u[:4], u[:-4]], axis=0)
        ud = jnp.concatenate([u[4:], u[-4:]], axis=0)
        oh = jnp.maximum(jnp.maximum(u, uu * mhl), ud * mhr)
        left = jnp.concatenate([oh[:, :1], oh[:, :-1]], axis=1)
        right = jnp.concatenate([oh[:, 1:], oh[:, -1:]], axis=1)
        a = jnp.maximum(left, oh)
        b = jnp.maximum(oh, right)
        c = jnp.maximum(a, right)
        lhs = jnp.concatenate([a, c, b], axis=1).astype(jnp.bfloat16)
        o_ref[g] = jnp.dot(lhs, e768_ref[...],
                           preferred_element_type=jnp.float32)


def kernel(inputs):
    bsz = inputs.shape[0]
    nmaps = bsz * 3
    v = jnp.transpose(inputs[..., :3], (0, 3, 1, 2)).reshape(nmaps, _HF, _HF)
    i = jnp.arange(_HF, dtype=jnp.int32)
    p = jnp.arange(_HO, dtype=jnp.int32)
    q, r = p // 4, p % 4
    ehm = (q[:, None] == i[None, :]).astype(jnp.bfloat16)
    src = jnp.where(r == 0, q, jnp.where(r == 3, 512 + q, 256 + q))
    e768 = (jnp.arange(768, dtype=jnp.int32)[:, None] == src[None, :]
            ).astype(jnp.bfloat16)
    mhl = jnp.tile((r <= 2).astype(jnp.float32)[:, None], (1, 128))
    mhr = jnp.tile((r >= 1).astype(jnp.float32)[:, None], (1, 128))

    out = pl.pallas_call(
        _rf_body,
        grid=((nmaps + _G - 1) // _G,),
        in_specs=[
            pl.BlockSpec((_G, _HF, _HF), lambda m: (m, 0, 0)),
            pl.BlockSpec((_HO, _HF), lambda m: (0, 0)),
            pl.BlockSpec((768, _HO), lambda m: (0, 0)),
            pl.BlockSpec((_HO, 128), lambda m: (0, 0)),
            pl.BlockSpec((_HO, 128), lambda m: (0, 0)),
        ],
        out_specs=pl.BlockSpec((_G, _HO, _HO), lambda m: (m, 0, 0)),
        out_shape=jax.ShapeDtypeStruct((nmaps, _HO, _HO), jnp.float32),
        compiler_params=pltpu.CompilerParams(
            dimension_semantics=("parallel",),
        ),
    )(v, ehm, e768, mhl, mhr)
    return out.reshape(bsz, 3, _HO, _HO)


# speedup vs baseline: 1.0394x; 1.0321x over previous
import jax
import jax.numpy as jnp
from jax.experimental import pallas as pl
from jax.experimental.pallas import tpu as pltpu

_HF = 256
_HO = _HF * 4


_G = 4


def _rf_body(v_ref, eh_ref, e768_ref, mhl_ref, mhr_ref, o_ref):
    mhl = jnp.concatenate([mhl_ref[...], mhl_ref[...]], axis=1)
    mhr = jnp.concatenate([mhr_ref[...], mhr_ref[...]], axis=1)
    for g in range(_G):
        v = jnp.maximum(v_ref[g], jnp.bfloat16(0.0))
        u = jnp.dot(eh_ref[...], v, preferred_element_type=jnp.float32)
        uu = jnp.concatenate([u[:4], u[:-4]], axis=0)
        ud = jnp.concatenate([u[4:], u[-4:]], axis=0)
        oh = jnp.maximum(jnp.maximum(u, uu * mhl), ud * mhr)
        left = jnp.concatenate([oh[:, :1], oh[:, :-1]], axis=1)
        right = jnp.concatenate([oh[:, 1:], oh[:, -1:]], axis=1)
        a = jnp.maximum(left, oh)
        b = jnp.maximum(oh, right)
        c = jnp.maximum(a, right)
        lhs = jnp.concatenate([a, c, b], axis=1).astype(jnp.bfloat16)
        o_ref[g] = jnp.dot(lhs, e768_ref[...],
                           preferred_element_type=jnp.float32)


def kernel(inputs):
    bsz = inputs.shape[0]
    nmaps = bsz * 3
    v = jnp.transpose(inputs[..., :3], (0, 3, 1, 2)).reshape(
        nmaps, _HF, _HF).astype(jnp.bfloat16)
    i = jnp.arange(_HF, dtype=jnp.int32)
    p = jnp.arange(_HO, dtype=jnp.int32)
    q, r = p // 4, p % 4
    ehm = (q[:, None] == i[None, :]).astype(jnp.bfloat16)
    src = jnp.where(r == 0, q, jnp.where(r == 3, 512 + q, 256 + q))
    e768 = (jnp.arange(768, dtype=jnp.int32)[:, None] == src[None, :]
            ).astype(jnp.bfloat16)
    mhl = jnp.tile((r <= 2).astype(jnp.float32)[:, None], (1, 128))
    mhr = jnp.tile((r >= 1).astype(jnp.float32)[:, None], (1, 128))

    out = pl.pallas_call(
        _rf_body,
        grid=((nmaps + _G - 1) // _G,),
        in_specs=[
            pl.BlockSpec((_G, _HF, _HF), lambda m: (m, 0, 0)),
            pl.BlockSpec((_HO, _HF), lambda m: (0, 0)),
            pl.BlockSpec((768, _HO), lambda m: (0, 0)),
            pl.BlockSpec((_HO, 128), lambda m: (0, 0)),
            pl.BlockSpec((_HO, 128), lambda m: (0, 0)),
        ],
        out_specs=pl.BlockSpec((_G, _HO, _HO), lambda m: (m, 0, 0)),
        out_shape=jax.ShapeDtypeStruct((nmaps, _HO, _HO), jnp.float32),
        compiler_params=pltpu.CompilerParams(
            dimension_semantics=("parallel",),
        ),
    )(v, ehm, e768, mhl, mhr)
    return out.reshape(bsz, 3, _HO, _HO)
